# baseline (speedup 1.0000x reference)
import jax
import jax.numpy as jnp
from jax.experimental import pallas as pl
from jax.experimental.pallas import tpu as pltpu

B, S, H, D = 16, 4096, 16, 64
Q = 16
S_BLK = 512


def _fused_kernel(pos_ref, k_ref, v_ref, kval_ref, vval_ref, ko_ref, vo_ref):
    si = pl.program_id(1)
    base = si * S_BLK
    ko_ref[0] = jnp.swapaxes(k_ref[0], 0, 1)
    vo_ref[0] = jnp.swapaxes(v_ref[0], 0, 1)
    for q in range(Q):
        local = pos_ref[q] - base

        @pl.when((local >= 0) & (local < S_BLK))
        def _():
            ko_ref[0, :, pl.ds(local, 1), :] = kval_ref[0, q][:, None, :]
            vo_ref[0, :, pl.ds(local, 1), :] = vval_ref[0, q][:, None, :]


def kernel(past_k_caches, past_v_caches, input_pos, k_val, v_val):
    pos = input_pos.astype(jnp.int32)
    grid_spec = pltpu.PrefetchScalarGridSpec(
        num_scalar_prefetch=1,
        grid=(B, S // S_BLK),
        in_specs=[
            pl.BlockSpec((1, S_BLK, H, D), lambda b, si, pos: (b, si, 0, 0)),
            pl.BlockSpec((1, S_BLK, H, D), lambda b, si, pos: (b, si, 0, 0)),
            pl.BlockSpec((1, Q, H, D), lambda b, si, pos: (b, 0, 0, 0)),
            pl.BlockSpec((1, Q, H, D), lambda b, si, pos: (b, 0, 0, 0)),
        ],
        out_specs=[
            pl.BlockSpec((1, H, S_BLK, D), lambda b, si, pos: (b, 0, si, 0)),
            pl.BlockSpec((1, H, S_BLK, D), lambda b, si, pos: (b, 0, si, 0)),
        ],
    )
    out_shape = [
        jax.ShapeDtypeStruct((B, H, S, D), jnp.float32),
        jax.ShapeDtypeStruct((B, H, S, D), jnp.float32),
    ]
    k_out, v_out = pl.pallas_call(
        _fused_kernel,
        grid_spec=grid_spec,
        out_shape=out_shape,
        compiler_params=pltpu.CompilerParams(
            dimension_semantics=("parallel", "parallel"),
        ),
    )(pos, past_k_caches, past_v_caches, k_val, v_val)
    return (k_out, v_out)
